# baseline (speedup 1.0000x reference)
import functools

import jax
import jax.numpy as jnp
from jax import lax
from jax.experimental import pallas as pl
from jax.experimental.pallas import tpu as pltpu
from jax.experimental.pallas import tpu_sc as plsc

_NW = 32
_LANES = 16
_BLK = 128
_CBLK = 25


def _bc_f32(v):
    return plsc.bitcast(v, jnp.float32)


@jax.jit
def _sc_energy(packed, ei3, at3):
    n_nodes = packed.shape[0]
    n_blocks = ei3.shape[0]
    n_chunks = n_blocks // _CBLK
    mesh = plsc.VectorSubcoreMesh(core_axis_name="c", subcore_axis_name="s")

    @functools.partial(
        pl.kernel,
        mesh=mesh,
        out_type=jax.ShapeDtypeStruct((_NW * _LANES,), jnp.float32),
        compiler_params=pltpu.CompilerParams(needs_layout_passes=False),
        scratch_types=[
            pltpu.VMEM((n_nodes,), jnp.int32),
            pltpu.VMEM((2, _CBLK, 2, _BLK), jnp.int32),
            pltpu.VMEM((2, _CBLK, 2, _BLK), jnp.float32),
            pltpu.VMEM((_LANES,), jnp.float32),
            pltpu.SemaphoreType.DMA((2,)),
            pltpu.SemaphoreType.DMA,
        ],
    )
    def launch(packed_hbm, ei_hbm, at_hbm, out_hbm, table_v, ei_v, at_v,
               acc_v, sem, tsem):
        wid = lax.axis_index("s") * 2 + lax.axis_index("c")
        my_chunks = (n_chunks - wid + (_NW - 1)) // _NW

        def issue(t, slot):
            blk0 = (wid + t * _NW) * _CBLK
            pltpu.make_async_copy(ei_hbm.at[pl.ds(blk0, _CBLK)],
                                  ei_v.at[slot], sem.at[slot]).start()
            pltpu.make_async_copy(at_hbm.at[pl.ds(blk0, _CBLK)],
                                  at_v.at[slot], sem.at[slot]).start()

        tbl = pltpu.make_async_copy(packed_hbm, table_v, tsem)
        tbl.start()
        issue(0, 0)
        tbl.wait()

        def chunk_body(t, acc):
            slot = t & 1
            pltpu.make_async_copy(ei_hbm.at[pl.ds(0, _CBLK)],
                                  ei_v.at[slot], sem.at[slot]).wait()
            pltpu.make_async_copy(at_hbm.at[pl.ds(0, _CBLK)],
                                  at_v.at[slot], sem.at[slot]).wait()

            @pl.when(t + 1 < my_chunks)
            def _():
                issue(t + 1, 1 - slot)

            @plsc.parallel_loop(0, _CBLK * (_BLK // _LANES),
                                unroll=20, carry=acc)
            def inner(i, acc):
                b = i >> 3
                u = i & 7
                if True:
                    sl = pl.ds(u * _LANES, _LANES)
                    i0 = ei_v[slot, b, 0, sl]
                    i1 = ei_v[slot, b, 1, sl]
                    lv = at_v[slot, b, 0, sl]
                    kv = at_v[slot, b, 1, sl]
                    w0 = plsc.load_gather(table_v, [i0])
                    w1 = plsc.load_gather(table_v, [i1])
                    d = (plsc.bitcast(w0, jnp.bfloat16)
                         - plsc.bitcast(w1, jnp.bfloat16))
                    dx, dy = plsc.unpack(d, format=plsc.PackFormat.INTERLEAVED)
                    s = dx * dx + dy * dy
                    m = (jnp.int32(0x5F3759DF)
                         - (plsc.bitcast(s, jnp.int32) >> 1))
                    r = _bc_f32(m)
                    h = s * 0.5
                    r = r * (1.5008909 - h * r * r)
                    sq2 = (s + s) * r
                    e = kv * (s + lv * lv - sq2 * lv)
                    return acc + e

            return inner

        acc = lax.fori_loop(0, my_chunks, chunk_body,
                            jnp.zeros((_LANES,), jnp.float32))
        acc_v[...] = acc
        pltpu.sync_copy(acc_v, out_hbm.at[pl.ds(wid * _LANES, _LANES)])

    return launch(packed, ei3, at3)


def kernel(p, edge_index, edge_attr):
    n_edges = edge_index.shape[1]
    nb = n_edges // _BLK
    xb = lax.bitcast_convert_type(p[:, 0].astype(jnp.bfloat16), jnp.uint16)
    yb = lax.bitcast_convert_type(p[:, 1].astype(jnp.bfloat16), jnp.uint16)
    packed = lax.bitcast_convert_type(
        xb.astype(jnp.uint32) | (yb.astype(jnp.uint32) << 16), jnp.int32)
    ei3 = edge_index.astype(jnp.int32).reshape(2, nb, _BLK).transpose(1, 0, 2)
    at3 = edge_attr.reshape(nb, _BLK, 2).transpose(0, 2, 1)
    partial = _sc_energy(packed, ei3, at3)
    return 0.5 * jnp.sum(partial)


# speedup vs baseline: 1.3499x; 1.3499x over previous
import functools

import jax
import jax.numpy as jnp
from jax import lax
from jax.experimental import pallas as pl
from jax.experimental.pallas import tpu as pltpu
from jax.experimental.pallas import tpu_sc as plsc

_NW = 32
_LANES = 16
_BLK = 128
_CBLK = 20


def _bc_f32(v):
    return plsc.bitcast(v, jnp.float32)


@jax.jit
def _sc_energy(packed, ei3, at3):
    n_nodes = packed.shape[0]
    n_blocks = ei3.shape[0]
    n_chunks = n_blocks // _CBLK
    mesh = plsc.VectorSubcoreMesh(core_axis_name="c", subcore_axis_name="s")

    @functools.partial(
        pl.kernel,
        mesh=mesh,
        out_type=jax.ShapeDtypeStruct((_NW * _LANES,), jnp.float32),
        compiler_params=pltpu.CompilerParams(needs_layout_passes=False),
        scratch_types=[
            pltpu.VMEM((n_nodes,), jnp.int32),
            pltpu.VMEM((2, _CBLK, 2, _BLK), jnp.int32),
            pltpu.VMEM((2, _CBLK, 2, _BLK), jnp.float32),
            pltpu.VMEM((_LANES,), jnp.float32),
            pltpu.SemaphoreType.DMA((2,)),
            pltpu.SemaphoreType.DMA,
        ],
    )
    def launch(packed_hbm, ei_hbm, at_hbm, out_hbm, table_v, ei_v, at_v,
               acc_v, sem, tsem):
        wid = lax.axis_index("s") * 2 + lax.axis_index("c")
        my_chunks = (n_chunks - wid + (_NW - 1)) // _NW

        def issue(t, slot):
            blk0 = (wid + t * _NW) * _CBLK
            pltpu.make_async_copy(ei_hbm.at[pl.ds(blk0, _CBLK)],
                                  ei_v.at[slot], sem.at[slot]).start()
            pltpu.make_async_copy(at_hbm.at[pl.ds(blk0, _CBLK)],
                                  at_v.at[slot], sem.at[slot]).start()

        tbl = pltpu.make_async_copy(packed_hbm, table_v, tsem)
        tbl.start()
        issue(0, 0)
        tbl.wait()

        def chunk_body(t, acc):
            slot = t & 1
            pltpu.make_async_copy(ei_hbm.at[pl.ds(0, _CBLK)],
                                  ei_v.at[slot], sem.at[slot]).wait()
            pltpu.make_async_copy(at_hbm.at[pl.ds(0, _CBLK)],
                                  at_v.at[slot], sem.at[slot]).wait()

            @pl.when(t + 1 < my_chunks)
            def _():
                issue(t + 1, 1 - slot)

            @plsc.parallel_loop(0, _CBLK * (_BLK // _LANES),
                                unroll=16, carry=acc)
            def inner(i, acc):
                b = i >> 3
                u = i & 7
                if True:
                    sl = pl.ds(u * _LANES, _LANES)
                    i0 = ei_v[slot, b, 0, sl]
                    i1 = ei_v[slot, b, 1, sl]
                    lv = at_v[slot, b, 0, sl]
                    kv = at_v[slot, b, 1, sl]
                    w0 = plsc.load_gather(table_v, [i0])
                    w1 = plsc.load_gather(table_v, [i1])
                    d = (plsc.bitcast(w0, jnp.bfloat16)
                         - plsc.bitcast(w1, jnp.bfloat16))
                    dx, dy = plsc.unpack(d, format=plsc.PackFormat.INTERLEAVED)
                    s = dx * dx + dy * dy
                    m = (jnp.int32(0x5F3759DF)
                         - (plsc.bitcast(s, jnp.int32) >> 1))
                    r = _bc_f32(m)
                    h = s * 0.5
                    r = r * (1.5008909 - h * r * r)
                    sq2 = (s + s) * r
                    e = kv * (s + lv * lv - sq2 * lv)
                    return acc + e

            return inner

        acc = lax.fori_loop(0, my_chunks, chunk_body,
                            jnp.zeros((_LANES,), jnp.float32))
        acc_v[...] = acc
        pltpu.sync_copy(acc_v, out_hbm.at[pl.ds(wid * _LANES, _LANES)])

    return launch(packed, ei3, at3)


def kernel(p, edge_index, edge_attr):
    n_edges = edge_index.shape[1]
    nb = n_edges // _BLK
    xb = lax.bitcast_convert_type(p[:, 0].astype(jnp.bfloat16), jnp.uint16)
    yb = lax.bitcast_convert_type(p[:, 1].astype(jnp.bfloat16), jnp.uint16)
    packed = lax.bitcast_convert_type(
        xb.astype(jnp.uint32) | (yb.astype(jnp.uint32) << 16), jnp.int32)
    ei3 = edge_index.astype(jnp.int32).reshape(2, nb, _BLK).transpose(1, 0, 2)
    at3 = edge_attr.reshape(nb, _BLK, 2).transpose(0, 2, 1)
    partial = _sc_energy(packed, ei3, at3)
    return 0.5 * jnp.sum(partial)
